# baseline (speedup 1.0000x reference)
import functools

import jax
import jax.numpy as jnp
from jax import lax
from jax.experimental import pallas as pl
from jax.experimental.pallas import tpu as pltpu
from jax.experimental.pallas import tpu_sc as plsc

N = 10000
E = 320000
D = 128
H = 128
C = 40

W1EXT = 144
W2EXT = 48

NC = 2
NS = 16
NW = NC * NS
K = 128
G2 = 80
SB = 10
NSB = G2 // SB
E_PAD = NW * G2 * K
N_PAD = 10112
ROWS_PER_TILE = N_PAD // NS

BLK = 1000


def _make_sc_aggregate(width):
    mesh = plsc.VectorSubcoreMesh(
        core_axis_name="c", subcore_axis_name="s", num_cores=NC, num_subcores=NS
    )

    @functools.partial(
        pl.kernel,
        mesh=mesh,
        compiler_params=pltpu.CompilerParams(use_tc_tiling_on_sc=False),
        out_type=jax.ShapeDtypeStruct((NC, N_PAD, width), jnp.float32),
        scratch_types=[
            pltpu.VMEM((SB, K), jnp.int32),
            pltpu.VMEM((SB, K), jnp.int32),
            pltpu.VMEM((K, width), jnp.float32),
            pltpu.VMEM((K, width), jnp.float32),
            pltpu.VMEM_SHARED((N_PAD, width), jnp.float32),
            pltpu.SemaphoreType.DMA,
            pltpu.SemaphoreType.DMA,
        ],
    )
    def agg(h_hbm, src_hbm, dst_hbm, zeros_hbm, out_hbm,
            idx_s, idx_d, rows_a, rows_b, acc, sem_a, sem_b):
        c = lax.axis_index("c")
        s = lax.axis_index("s")
        wid = c * NS + s
        r0 = s * ROWS_PER_TILE

        pltpu.sync_copy(zeros_hbm.at[pl.ds(r0, ROWS_PER_TILE)],
                        acc.at[pl.ds(r0, ROWS_PER_TILE)])
        plsc.subcore_barrier()

        row0 = wid * G2

        @pl.loop(0, NSB)
        def _(b):
            blk = row0 + b * SB
            pltpu.sync_copy(src_hbm.at[pl.ds(blk, SB)], idx_s)
            pltpu.sync_copy(dst_hbm.at[pl.ds(blk, SB)], idx_d)
            pltpu.async_copy(h_hbm.at[idx_s.at[0]], rows_a, sem_a)
            pltpu.async_copy(h_hbm.at[idx_s.at[1]], rows_b, sem_b)

            @pl.loop(0, SB - 2, step=2)
            def _(j):
                pltpu.make_async_copy(h_hbm.at[idx_s.at[j]], rows_a, sem_a).wait()
                pltpu.sync_copy(rows_a, acc.at[idx_d.at[j]], add=True)
                pltpu.async_copy(h_hbm.at[idx_s.at[j + 2]], rows_a, sem_a)
                pltpu.make_async_copy(h_hbm.at[idx_s.at[j + 1]], rows_b, sem_b).wait()
                pltpu.sync_copy(rows_b, acc.at[idx_d.at[j + 1]], add=True)
                pltpu.async_copy(h_hbm.at[idx_s.at[j + 3]], rows_b, sem_b)

            pltpu.make_async_copy(h_hbm.at[idx_s.at[SB - 2]], rows_a, sem_a).wait()
            pltpu.sync_copy(rows_a, acc.at[idx_d.at[SB - 2]], add=True)
            pltpu.make_async_copy(h_hbm.at[idx_s.at[SB - 1]], rows_b, sem_b).wait()
            pltpu.sync_copy(rows_b, acc.at[idx_d.at[SB - 1]], add=True)

        plsc.subcore_barrier()
        pltpu.sync_copy(acc.at[pl.ds(r0, ROWS_PER_TILE)],
                        out_hbm.at[c, pl.ds(r0, ROWS_PER_TILE)])

    return agg


_agg1 = _make_sc_aggregate(W1EXT)
_agg2 = _make_sc_aggregate(W2EXT)


def _mm1_body(x_ref, w_ref, o_ref):
    h = jnp.dot(x_ref[...], w_ref[...],
                preferred_element_type=jnp.float32,
                precision=lax.Precision.HIGHEST)
    o_ref[:, :D] = h
    col = lax.broadcasted_iota(jnp.int32, (BLK, W1EXT - D), 1)
    o_ref[:, D:] = jnp.where(col == 0, 1.0, 0.0)


def _mm1(x, w1):
    return pl.pallas_call(
        _mm1_body,
        grid=(N // BLK,),
        in_specs=[
            pl.BlockSpec((BLK, D), lambda i: (i, 0)),
            pl.BlockSpec((D, H), lambda i: (0, 0)),
        ],
        out_specs=pl.BlockSpec((BLK, W1EXT), lambda i: (i, 0)),
        out_shape=jax.ShapeDtypeStruct((N, W1EXT), jnp.float32),
    )(x, w1)


def _fin1_body(a_ref, b1_ref, w2_ref, h2_ref, rdeg_ref):
    su = a_ref[0] + a_ref[1]
    deg = jnp.maximum(su[:, D:D + 1], 1.0)
    rdeg = 1.0 / deg
    h1 = jnp.maximum(su[:, :D] * rdeg + b1_ref[...], 0.0)
    h2_ref[...] = jnp.dot(h1, w2_ref[...],
                          preferred_element_type=jnp.float32,
                          precision=lax.Precision.HIGHEST)
    rdeg_ref[...] = rdeg


def _fin1(acc, b1, w2p):
    return pl.pallas_call(
        _fin1_body,
        grid=(N // BLK,),
        in_specs=[
            pl.BlockSpec((NC, BLK, W1EXT), lambda i: (0, i, 0)),
            pl.BlockSpec((1, H), lambda i: (0, 0)),
            pl.BlockSpec((H, W2EXT), lambda i: (0, 0)),
        ],
        out_specs=[
            pl.BlockSpec((BLK, W2EXT), lambda i: (i, 0)),
            pl.BlockSpec((BLK, 1), lambda i: (i, 0)),
        ],
        out_shape=[
            jax.ShapeDtypeStruct((N, W2EXT), jnp.float32),
            jax.ShapeDtypeStruct((N, 1), jnp.float32),
        ],
    )(acc, b1, w2p)


def _fin2_body(a_ref, rdeg_ref, b2_ref, o_ref):
    o_ref[...] = (a_ref[0] + a_ref[1]) * rdeg_ref[...] + b2_ref[...]


def _fin2(acc, rdeg, b2p):
    return pl.pallas_call(
        _fin2_body,
        grid=(N // BLK,),
        in_specs=[
            pl.BlockSpec((NC, BLK, W2EXT), lambda i: (0, i, 0)),
            pl.BlockSpec((BLK, 1), lambda i: (i, 0)),
            pl.BlockSpec((1, W2EXT), lambda i: (0, 0)),
        ],
        out_specs=pl.BlockSpec((BLK, W2EXT), lambda i: (i, 0)),
        out_shape=jax.ShapeDtypeStruct((N, W2EXT), jnp.float32),
    )(acc, rdeg, b2p)


def kernel(x, edge_index, W1, b1, W2, b2):
    src = jnp.concatenate(
        [edge_index[0], jnp.zeros((E_PAD - E,), jnp.int32)]
    ).reshape(E_PAD // K, K)
    dst = jnp.concatenate(
        [edge_index[1], jnp.full((E_PAD - E,), N_PAD - 1, jnp.int32)]
    ).reshape(E_PAD // K, K)

    hext = _mm1(x, W1)
    zeros1 = jnp.zeros((N_PAD, W1EXT), jnp.float32)
    acc1 = _agg1(hext, src, dst, zeros1)

    w2p = jnp.pad(W2, ((0, 0), (0, W2EXT - C)))
    h2, rdeg = _fin1(acc1, b1.reshape(1, H), w2p)

    zeros2 = jnp.zeros((N_PAD, W2EXT), jnp.float32)
    acc2 = _agg2(h2, src, dst, zeros2)

    b2p = jnp.pad(b2, (0, W2EXT - C)).reshape(1, W2EXT)
    out = _fin2(acc2, rdeg, b2p)
    return out[:, :C]


# speedup vs baseline: 2.3771x; 2.3771x over previous
import functools

import jax
import jax.numpy as jnp
from jax import lax
from jax.experimental import pallas as pl
from jax.experimental.pallas import tpu as pltpu
from jax.experimental.pallas import tpu_sc as plsc

N = 10000
E = 320000
D = 128
H = 128
C = 40

W1EXT = 144
W2EXT = 48

NC = 2
NS = 16
NW = NC * NS
K = 128
G2 = 80
SB = 10
NSB = G2 // SB
E_PAD = NW * G2 * K
N_PAD = 10112
ROWS_PER_TILE = N_PAD // NS

BLK = 1000


def _make_sc_aggregate(width):
    mesh = plsc.VectorSubcoreMesh(
        core_axis_name="c", subcore_axis_name="s", num_cores=NC, num_subcores=NS
    )

    @functools.partial(
        pl.kernel,
        mesh=mesh,
        compiler_params=pltpu.CompilerParams(use_tc_tiling_on_sc=False),
        out_type=jax.ShapeDtypeStruct((NC, N_PAD, width), jnp.float32),
        scratch_types=[
            pltpu.VMEM((SB, K), jnp.int32),
            pltpu.VMEM((SB, K), jnp.int32),
            pltpu.VMEM((K, width), jnp.float32),
            pltpu.VMEM((K, width), jnp.float32),
            pltpu.VMEM_SHARED((N_PAD, width), jnp.float32),
            pltpu.SemaphoreType.DMA,
            pltpu.SemaphoreType.DMA,
        ],
    )
    def agg(h_hbm, src_hbm, dst_hbm, zeros_hbm, out_hbm,
            idx_s, idx_d, rows_a, rows_b, acc, sem_a, sem_b):
        c = lax.axis_index("c")
        s = lax.axis_index("s")
        wid = c * NS + s
        r0 = s * ROWS_PER_TILE

        pltpu.sync_copy(zeros_hbm.at[pl.ds(r0, ROWS_PER_TILE)],
                        acc.at[pl.ds(r0, ROWS_PER_TILE)])
        plsc.subcore_barrier()

        row0 = wid * G2

        @pl.loop(0, NSB)
        def _(b):
            blk = row0 + b * SB
            pltpu.sync_copy(src_hbm.at[pl.ds(blk, SB)], idx_s)
            pltpu.sync_copy(dst_hbm.at[pl.ds(blk, SB)], idx_d)
            pltpu.async_copy(h_hbm.at[idx_s.at[0]], rows_a, sem_a)
            pltpu.async_copy(h_hbm.at[idx_s.at[1]], rows_b, sem_b)

            @pl.loop(0, SB - 2, step=2)
            def _(j):
                pltpu.make_async_copy(h_hbm.at[idx_s.at[j]], rows_a, sem_a).wait()
                pltpu.sync_copy(rows_a, acc.at[idx_d.at[j]], add=True)
                pltpu.async_copy(h_hbm.at[idx_s.at[j + 2]], rows_a, sem_a)
                pltpu.make_async_copy(h_hbm.at[idx_s.at[j + 1]], rows_b, sem_b).wait()
                pltpu.sync_copy(rows_b, acc.at[idx_d.at[j + 1]], add=True)
                pltpu.async_copy(h_hbm.at[idx_s.at[j + 3]], rows_b, sem_b)

            pltpu.make_async_copy(h_hbm.at[idx_s.at[SB - 2]], rows_a, sem_a).wait()
            pltpu.sync_copy(rows_a, acc.at[idx_d.at[SB - 2]], add=True)
            pltpu.make_async_copy(h_hbm.at[idx_s.at[SB - 1]], rows_b, sem_b).wait()
            pltpu.sync_copy(rows_b, acc.at[idx_d.at[SB - 1]], add=True)

        plsc.subcore_barrier()
        pltpu.sync_copy(acc.at[pl.ds(r0, ROWS_PER_TILE)],
                        out_hbm.at[c, pl.ds(r0, ROWS_PER_TILE)])

    return agg


_agg1 = _make_sc_aggregate(W1EXT)
_agg2 = _make_sc_aggregate(W2EXT)


def _mm1_body(x_ref, w_ref, o_ref):
    h = jnp.dot(x_ref[...], w_ref[...],
                preferred_element_type=jnp.float32,
                precision=lax.Precision.HIGHEST)
    o_ref[:, :D] = h
    col = lax.broadcasted_iota(jnp.int32, (BLK, W1EXT - D), 1)
    o_ref[:, D:] = jnp.where(col == 0, 1.0, 0.0)


def _mm1(x, w1):
    return pl.pallas_call(
        _mm1_body,
        grid=(N // BLK,),
        in_specs=[
            pl.BlockSpec((BLK, D), lambda i: (i, 0)),
            pl.BlockSpec((D, H), lambda i: (0, 0)),
        ],
        out_specs=pl.BlockSpec((BLK, W1EXT), lambda i: (i, 0)),
        out_shape=jax.ShapeDtypeStruct((N, W1EXT), jnp.float32),
    )(x, w1)


def _fin1_body(a_ref, b1_ref, w2_ref, h2_ref, rdeg_ref):
    su = a_ref[0] + a_ref[1]
    deg = jnp.maximum(su[:, D:D + 1], 1.0)
    rdeg = 1.0 / deg
    h1 = jnp.maximum(su[:, :D] * rdeg + b1_ref[...], 0.0)
    h2_ref[...] = jnp.dot(h1, w2_ref[...],
                          preferred_element_type=jnp.float32,
                          precision=lax.Precision.HIGHEST)
    rdeg_ref[...] = rdeg


def _fin1(acc, b1, w2p):
    return pl.pallas_call(
        _fin1_body,
        grid=(N // BLK,),
        in_specs=[
            pl.BlockSpec((NC, BLK, W1EXT), lambda i: (0, i, 0)),
            pl.BlockSpec((1, H), lambda i: (0, 0)),
            pl.BlockSpec((H, W2EXT), lambda i: (0, 0)),
        ],
        out_specs=[
            pl.BlockSpec((BLK, W2EXT), lambda i: (i, 0)),
            pl.BlockSpec((BLK, 1), lambda i: (i, 0)),
        ],
        out_shape=[
            jax.ShapeDtypeStruct((N, W2EXT), jnp.float32),
            jax.ShapeDtypeStruct((N, 1), jnp.float32),
        ],
    )(acc, b1, w2p)


def _fin2_body(a_ref, rdeg_ref, b2_ref, o_ref):
    o_ref[...] = (a_ref[0] + a_ref[1]) * rdeg_ref[...] + b2_ref[...]


def _fin2(acc, rdeg, b2p):
    return pl.pallas_call(
        _fin2_body,
        grid=(N // BLK,),
        in_specs=[
            pl.BlockSpec((NC, BLK, W2EXT), lambda i: (0, i, 0)),
            pl.BlockSpec((BLK, 1), lambda i: (i, 0)),
            pl.BlockSpec((1, W2EXT), lambda i: (0, 0)),
        ],
        out_specs=pl.BlockSpec((BLK, W2EXT), lambda i: (i, 0)),
        out_shape=jax.ShapeDtypeStruct((N, W2EXT), jnp.float32),
    )(acc, rdeg, b2p)


def kernel(x, edge_index, W1, b1, W2, b2):
    pidx = jnp.arange(E_PAD - E, dtype=jnp.int32)
    src = jnp.concatenate([edge_index[0], pidx % N]).reshape(E_PAD // K, K)
    dst = jnp.concatenate(
        [edge_index[1], N + pidx % (N_PAD - N)]
    ).reshape(E_PAD // K, K)

    hext = _mm1(x, W1)
    zeros1 = jnp.zeros((N_PAD, W1EXT), jnp.float32)
    acc1 = _agg1(hext, src, dst, zeros1)

    w2p = jnp.pad(W2, ((0, 0), (0, W2EXT - C)))
    h2, rdeg = _fin1(acc1, b1.reshape(1, H), w2p)

    zeros2 = jnp.zeros((N_PAD, W2EXT), jnp.float32)
    acc2 = _agg2(h2, src, dst, zeros2)

    b2p = jnp.pad(b2, (0, W2EXT - C)).reshape(1, W2EXT)
    out = _fin2(acc2, rdeg, b2p)
    return out[:, :C]


# speedup vs baseline: 2.5294x; 1.0641x over previous
import functools

import jax
import jax.numpy as jnp
from jax import lax
from jax.experimental import pallas as pl
from jax.experimental.pallas import tpu as pltpu
from jax.experimental.pallas import tpu_sc as plsc

N = 10000
E = 320000
D = 128
H = 128
C = 40

HW = 72
W2EXT = 48

NC = 2
NS = 16
NW = NC * NS
K = 128
NCHUNK = 2560
E_PAD = NCHUNK * K
SB = 20
NSLOT = 4
N_PAD = 10112
ROWS_PER_TILE = N_PAD // NS

BLK = 1000


def _pipeline_chunks(h_view, acc, idx_s, idx_d, rows, gsem, ssem, nsb, row0):
    @pl.loop(0, nsb)
    def _(b):
        blk = row0 + b * SB
        pltpu.sync_copy(idx_s[1].at[pl.ds(blk, SB)], idx_s[0])
        pltpu.sync_copy(idx_d[1].at[pl.ds(blk, SB)], idx_d[0])
        s_v, d_v = idx_s[0], idx_d[0]

        for t in range(NSLOT):
            pltpu.async_copy(h_view.at[s_v.at[t]], rows[t], gsem[t])
        for t in range(NSLOT):
            pltpu.make_async_copy(h_view.at[s_v.at[t]], rows[t], gsem[t]).wait()
            pltpu.async_copy(rows[t], acc.at[d_v.at[t]], ssem[t], add=True)

        @pl.loop(NSLOT, SB, step=NSLOT)
        def _(j):
            for t in range(NSLOT):
                pltpu.make_async_copy(
                    rows[t], acc.at[d_v.at[j + t - NSLOT]], ssem[t]).wait()
                pltpu.async_copy(h_view.at[s_v.at[j + t]], rows[t], gsem[t])
            for t in range(NSLOT):
                pltpu.make_async_copy(
                    h_view.at[s_v.at[j + t]], rows[t], gsem[t]).wait()
                pltpu.async_copy(rows[t], acc.at[d_v.at[j + t]], ssem[t],
                                 add=True)

        for t in range(NSLOT):
            pltpu.make_async_copy(
                rows[t], acc.at[d_v.at[SB + t - NSLOT]], ssem[t]).wait()


def _make_sc_agg(width, feature_split):
    mesh = plsc.VectorSubcoreMesh(
        core_axis_name="c", subcore_axis_name="s", num_cores=NC, num_subcores=NS
    )
    h_shape = (NC, N, width) if feature_split else (N, width)

    @functools.partial(
        pl.kernel,
        mesh=mesh,
        compiler_params=pltpu.CompilerParams(use_tc_tiling_on_sc=False),
        out_type=jax.ShapeDtypeStruct((NC, N_PAD, width), jnp.float32),
        scratch_types=[
            pltpu.VMEM((SB, K), jnp.int32),
            pltpu.VMEM((SB, K), jnp.int32),
            pltpu.VMEM((K, width), jnp.float32),
            pltpu.VMEM((K, width), jnp.float32),
            pltpu.VMEM((K, width), jnp.float32),
            pltpu.VMEM((K, width), jnp.float32),
            pltpu.VMEM_SHARED((N_PAD, width), jnp.float32),
            pltpu.SemaphoreType.DMA,
            pltpu.SemaphoreType.DMA,
            pltpu.SemaphoreType.DMA,
            pltpu.SemaphoreType.DMA,
            pltpu.SemaphoreType.DMA,
            pltpu.SemaphoreType.DMA,
            pltpu.SemaphoreType.DMA,
            pltpu.SemaphoreType.DMA,
        ],
    )
    def agg(h_hbm, src_hbm, dst_hbm, zeros_hbm, out_hbm,
            idx_s, idx_d, r0b, r1b, r2b, r3b, acc,
            g0, g1, g2, g3, s0, s1, s2, s3):
        rows = [r0b, r1b, r2b, r3b]
        gsem = [g0, g1, g2, g3]
        ssem = [s0, s1, s2, s3]
        c = lax.axis_index("c")
        s = lax.axis_index("s")
        r0 = s * ROWS_PER_TILE

        pltpu.sync_copy(zeros_hbm.at[pl.ds(r0, ROWS_PER_TILE)],
                        acc.at[pl.ds(r0, ROWS_PER_TILE)])
        plsc.subcore_barrier()

        if feature_split:
            h_view = h_hbm.at[c]
            chunks_per_tile = NCHUNK // NS
            row0 = s * chunks_per_tile
        else:
            h_view = h_hbm
            chunks_per_tile = NCHUNK // NW
            row0 = (c * NS + s) * chunks_per_tile

        _pipeline_chunks(h_view, acc, (idx_s, src_hbm), (idx_d, dst_hbm),
                         rows, gsem, ssem, chunks_per_tile // SB, row0)

        plsc.subcore_barrier()
        pltpu.sync_copy(acc.at[pl.ds(r0, ROWS_PER_TILE)],
                        out_hbm.at[c, pl.ds(r0, ROWS_PER_TILE)])

    return agg


_agg1 = _make_sc_agg(HW, feature_split=True)
_agg2 = _make_sc_agg(W2EXT, feature_split=False)


def _mm1_body(x_ref, w_ref, o0_ref, o1_ref):
    h = jnp.dot(x_ref[...], w_ref[...],
                preferred_element_type=jnp.float32,
                precision=lax.Precision.HIGHEST)
    o0_ref[...] = h[:, :HW]
    col = lax.broadcasted_iota(jnp.int32, (BLK, 2 * HW - D), 1)
    ones = jnp.where(col == D - HW, 1.0, 0.0)
    o1_ref[:, :D - HW] = h[:, HW:]
    o1_ref[:, D - HW:] = ones


def _mm1(x, w1):
    return pl.pallas_call(
        _mm1_body,
        grid=(N // BLK,),
        in_specs=[
            pl.BlockSpec((BLK, D), lambda i: (i, 0)),
            pl.BlockSpec((D, H), lambda i: (0, 0)),
        ],
        out_specs=[
            pl.BlockSpec((BLK, HW), lambda i: (i, 0)),
            pl.BlockSpec((BLK, HW), lambda i: (i, 0)),
        ],
        out_shape=[
            jax.ShapeDtypeStruct((N, HW), jnp.float32),
            jax.ShapeDtypeStruct((N, HW), jnp.float32),
        ],
    )(x, w1)


def _fin1_body(a_ref, b1_ref, w2_ref, h2_ref, rdeg_ref):
    half0 = a_ref[0]
    half1 = a_ref[1]
    su = jnp.concatenate([half0, half1[:, :D - HW]], axis=1)
    deg = jnp.maximum(half1[:, D - HW:D - HW + 1], 1.0)
    rdeg = 1.0 / deg
    h1 = jnp.maximum(su * rdeg + b1_ref[...], 0.0)
    h2_ref[...] = jnp.dot(h1, w2_ref[...],
                          preferred_element_type=jnp.float32,
                          precision=lax.Precision.HIGHEST)
    rdeg_ref[...] = rdeg


def _fin1(acc, b1, w2p):
    return pl.pallas_call(
        _fin1_body,
        grid=(N // BLK,),
        in_specs=[
            pl.BlockSpec((NC, BLK, HW), lambda i: (0, i, 0)),
            pl.BlockSpec((1, H), lambda i: (0, 0)),
            pl.BlockSpec((H, W2EXT), lambda i: (0, 0)),
        ],
        out_specs=[
            pl.BlockSpec((BLK, W2EXT), lambda i: (i, 0)),
            pl.BlockSpec((BLK, 1), lambda i: (i, 0)),
        ],
        out_shape=[
            jax.ShapeDtypeStruct((N, W2EXT), jnp.float32),
            jax.ShapeDtypeStruct((N, 1), jnp.float32),
        ],
    )(acc, b1, w2p)


def _fin2_body(a_ref, rdeg_ref, b2_ref, o_ref):
    o = (a_ref[0] + a_ref[1]) * rdeg_ref[...] + b2_ref[...]
    o_ref[...] = o[:, :C]


def _fin2(acc, rdeg, b2p):
    return pl.pallas_call(
        _fin2_body,
        grid=(N // BLK,),
        in_specs=[
            pl.BlockSpec((NC, BLK, W2EXT), lambda i: (0, i, 0)),
            pl.BlockSpec((BLK, 1), lambda i: (i, 0)),
            pl.BlockSpec((1, W2EXT), lambda i: (0, 0)),
        ],
        out_specs=pl.BlockSpec((BLK, C), lambda i: (i, 0)),
        out_shape=jax.ShapeDtypeStruct((N, C), jnp.float32),
    )(acc, rdeg, b2p)


def kernel(x, edge_index, W1, b1, W2, b2):
    pidx = jnp.arange(E_PAD - E, dtype=jnp.int32)
    src = jnp.concatenate([edge_index[0], pidx % N]).reshape(NCHUNK, K)
    dst = jnp.concatenate(
        [edge_index[1], N + pidx % (N_PAD - N)]
    ).reshape(NCHUNK, K)

    h0, h1 = _mm1(x, W1)
    hs = jnp.stack([h0, h1])
    zeros1 = jnp.zeros((N_PAD, HW), jnp.float32)
    acc1 = _agg1(hs, src, dst, zeros1)

    w2p = jnp.pad(W2, ((0, 0), (0, W2EXT - C)))
    h2, rdeg = _fin1(acc1, b1.reshape(1, H), w2p)

    zeros2 = jnp.zeros((N_PAD, W2EXT), jnp.float32)
    acc2 = _agg2(h2, src, dst, zeros2)

    b2p = jnp.pad(b2, (0, W2EXT - C)).reshape(1, W2EXT)
    return _fin2(acc2, rdeg, b2p)


# speedup vs baseline: 2.5299x; 1.0002x over previous
import functools

import jax
import jax.numpy as jnp
from jax import lax
from jax.experimental import pallas as pl
from jax.experimental.pallas import tpu as pltpu
from jax.experimental.pallas import tpu_sc as plsc

N = 10000
E = 320000
D = 128
H = 128
C = 40

HW = 72
W2EXT = 48

NC = 2
NS = 16
NW = NC * NS
K = 128
NCHUNK = 2560
E_PAD = NCHUNK * K
SB = 20
NSLOT = 4
N_PAD = 10112
ROWS_PER_TILE = N_PAD // NS

BLK = 1000


def _pipeline_chunks(h_view, acc, idx_s, idx_d, rows, gsem, ssem, nsb, row0):
    @pl.loop(0, nsb)
    def _(b):
        blk = row0 + b * SB
        pltpu.sync_copy(idx_s[1].at[pl.ds(blk, SB)], idx_s[0])
        pltpu.sync_copy(idx_d[1].at[pl.ds(blk, SB)], idx_d[0])
        s_v, d_v = idx_s[0], idx_d[0]

        for t in range(NSLOT):
            pltpu.async_copy(h_view.at[s_v.at[t]], rows[t], gsem[t])
        for t in range(NSLOT):
            pltpu.make_async_copy(h_view.at[s_v.at[t]], rows[t], gsem[t]).wait()
            pltpu.async_copy(rows[t], acc.at[d_v.at[t]], ssem[t], add=True)

        @pl.loop(NSLOT, SB, step=NSLOT)
        def _(j):
            for t in range(NSLOT):
                pltpu.make_async_copy(
                    rows[t], acc.at[d_v.at[j + t - NSLOT]], ssem[t]).wait()
                pltpu.async_copy(h_view.at[s_v.at[j + t]], rows[t], gsem[t])
            for t in range(NSLOT):
                pltpu.make_async_copy(
                    h_view.at[s_v.at[j + t]], rows[t], gsem[t]).wait()
                pltpu.async_copy(rows[t], acc.at[d_v.at[j + t]], ssem[t],
                                 add=True)

        for t in range(NSLOT):
            pltpu.make_async_copy(
                rows[t], acc.at[d_v.at[SB + t - NSLOT]], ssem[t]).wait()


def _make_sc_agg(width, feature_split):
    mesh = plsc.VectorSubcoreMesh(
        core_axis_name="c", subcore_axis_name="s", num_cores=NC, num_subcores=NS
    )
    h_shape = (NC, N, width) if feature_split else (N, width)

    @functools.partial(
        pl.kernel,
        mesh=mesh,
        compiler_params=pltpu.CompilerParams(use_tc_tiling_on_sc=False),
        out_type=jax.ShapeDtypeStruct((NC, N_PAD, width), jnp.float32),
        scratch_types=[
            pltpu.VMEM((SB, K), jnp.int32),
            pltpu.VMEM((SB, K), jnp.int32),
            pltpu.VMEM((K, width), jnp.float32),
            pltpu.VMEM((K, width), jnp.float32),
            pltpu.VMEM((K, width), jnp.float32),
            pltpu.VMEM((K, width), jnp.float32),
            pltpu.VMEM_SHARED((N_PAD, width), jnp.float32),
            pltpu.SemaphoreType.DMA,
            pltpu.SemaphoreType.DMA,
            pltpu.SemaphoreType.DMA,
            pltpu.SemaphoreType.DMA,
            pltpu.SemaphoreType.DMA,
            pltpu.SemaphoreType.DMA,
            pltpu.SemaphoreType.DMA,
            pltpu.SemaphoreType.DMA,
        ],
    )
    def agg(h_hbm, src_hbm, dst_hbm, zeros_hbm, out_hbm,
            idx_s, idx_d, r0b, r1b, r2b, r3b, acc,
            g0, g1, g2, g3, s0, s1, s2, s3):
        rows = [r0b, r1b, r2b, r3b]
        gsem = [g0, g1, g2, g3]
        ssem = [s0, s1, s2, s3]
        c = lax.axis_index("c")
        s = lax.axis_index("s")
        r0 = s * ROWS_PER_TILE

        pltpu.sync_copy(zeros_hbm.at[pl.ds(r0, ROWS_PER_TILE)],
                        acc.at[pl.ds(r0, ROWS_PER_TILE)])
        plsc.subcore_barrier()

        if feature_split:
            h_view = h_hbm.at[c]
            chunks_per_tile = NCHUNK // NS
            row0 = s * chunks_per_tile
        else:
            h_view = h_hbm
            chunks_per_tile = NCHUNK // NW
            row0 = (c * NS + s) * chunks_per_tile

        _pipeline_chunks(h_view, acc, (idx_s, src_hbm), (idx_d, dst_hbm),
                         rows, gsem, ssem, chunks_per_tile // SB, row0)

        plsc.subcore_barrier()
        pltpu.sync_copy(acc.at[pl.ds(r0, ROWS_PER_TILE)],
                        out_hbm.at[c, pl.ds(r0, ROWS_PER_TILE)])

    return agg


_agg1 = _make_sc_agg(HW, feature_split=True)
_agg2 = _make_sc_agg(W2EXT, feature_split=False)


def _mm1_body(x_ref, w_ref, o0_ref, o1_ref):
    h = jnp.dot(x_ref[...], w_ref[...],
                preferred_element_type=jnp.float32,
                precision=lax.Precision.HIGHEST)
    o0_ref[...] = h[:, :HW]
    col = lax.broadcasted_iota(jnp.int32, (BLK, 2 * HW - D), 1)
    ones = jnp.where(col == 0, 1.0, 0.0)
    o1_ref[:, :D - HW] = h[:, HW:]
    o1_ref[:, D - HW:] = ones


def _mm1(x, w1):
    return pl.pallas_call(
        _mm1_body,
        grid=(N // BLK,),
        in_specs=[
            pl.BlockSpec((BLK, D), lambda i: (i, 0)),
            pl.BlockSpec((D, H), lambda i: (0, 0)),
        ],
        out_specs=[
            pl.BlockSpec((BLK, HW), lambda i: (i, 0)),
            pl.BlockSpec((BLK, HW), lambda i: (i, 0)),
        ],
        out_shape=[
            jax.ShapeDtypeStruct((N, HW), jnp.float32),
            jax.ShapeDtypeStruct((N, HW), jnp.float32),
        ],
    )(x, w1)


def _fin1_body(a_ref, b1_ref, w2_ref, h2_ref, rdeg_ref):
    half0 = a_ref[0]
    half1 = a_ref[1]
    su = jnp.concatenate([half0, half1[:, :D - HW]], axis=1)
    deg = jnp.maximum(half1[:, D - HW:D - HW + 1], 1.0)
    rdeg = 1.0 / deg
    h1 = jnp.maximum(su * rdeg + b1_ref[...], 0.0)
    h2_ref[...] = jnp.dot(h1, w2_ref[...],
                          preferred_element_type=jnp.float32,
                          precision=lax.Precision.HIGHEST)
    rdeg_ref[...] = rdeg


def _fin1(acc, b1, w2p):
    return pl.pallas_call(
        _fin1_body,
        grid=(N // BLK,),
        in_specs=[
            pl.BlockSpec((NC, BLK, HW), lambda i: (0, i, 0)),
            pl.BlockSpec((1, H), lambda i: (0, 0)),
            pl.BlockSpec((H, W2EXT), lambda i: (0, 0)),
        ],
        out_specs=[
            pl.BlockSpec((BLK, W2EXT), lambda i: (i, 0)),
            pl.BlockSpec((BLK, 1), lambda i: (i, 0)),
        ],
        out_shape=[
            jax.ShapeDtypeStruct((N, W2EXT), jnp.float32),
            jax.ShapeDtypeStruct((N, 1), jnp.float32),
        ],
    )(acc, b1, w2p)


def _fin2_body(a_ref, rdeg_ref, b2_ref, o_ref):
    o = (a_ref[0] + a_ref[1]) * rdeg_ref[...] + b2_ref[...]
    o_ref[...] = o[:, :C]


def _fin2(acc, rdeg, b2p):
    return pl.pallas_call(
        _fin2_body,
        grid=(N // BLK,),
        in_specs=[
            pl.BlockSpec((NC, BLK, W2EXT), lambda i: (0, i, 0)),
            pl.BlockSpec((BLK, 1), lambda i: (i, 0)),
            pl.BlockSpec((1, W2EXT), lambda i: (0, 0)),
        ],
        out_specs=pl.BlockSpec((BLK, C), lambda i: (i, 0)),
        out_shape=jax.ShapeDtypeStruct((N, C), jnp.float32),
    )(acc, rdeg, b2p)


def kernel(x, edge_index, W1, b1, W2, b2):
    pidx = jnp.arange(E_PAD - E, dtype=jnp.int32)
    src = jnp.concatenate([edge_index[0], pidx % N]).reshape(NCHUNK, K)
    dst = jnp.concatenate(
        [edge_index[1], N + pidx % (N_PAD - N)]
    ).reshape(NCHUNK, K)

    h0, h1 = _mm1(x, W1)
    hs = jnp.stack([h0, h1])
    zeros1 = jnp.zeros((N_PAD, HW), jnp.float32)
    acc1 = _agg1(hs, src, dst, zeros1)

    w2p = jnp.pad(W2, ((0, 0), (0, W2EXT - C)))
    h2, rdeg = _fin1(acc1, b1.reshape(1, H), w2p)

    zeros2 = jnp.zeros((N_PAD, W2EXT), jnp.float32)
    acc2 = _agg2(h2, src, dst, zeros2)

    b2p = jnp.pad(b2, (0, W2EXT - C)).reshape(1, W2EXT)
    return _fin2(acc2, rdeg, b2p)
